# Initial kernel scaffold; baseline (speedup 1.0000x reference)
import functools

import jax
import jax.numpy as jnp
from jax import lax
from jax.experimental import pallas as pl
from jax.experimental.pallas import tpu as pltpu
from jax.experimental.pallas import tpu_sc as plsc

N = 10000
E = 320000
D = 128

NC = 2
NS = 16
NW = NC * NS
EPW = E // NW
CH = 80
NCHUNK = EPW // CH
RPT = N // NS


def _sc_partials(feature, src, dst):
    mesh = plsc.VectorSubcoreMesh(core_axis_name="c", subcore_axis_name="s")

    @functools.partial(
        pl.kernel,
        mesh=mesh,
        out_type=jax.ShapeDtypeStruct((NC * N, D), jnp.float32),
        scratch_types=[
            pltpu.VMEM_SHARED((N, D), jnp.float32),
            pltpu.VMEM((CH,), jnp.int32),
            pltpu.VMEM((CH,), jnp.int32),
            pltpu.VMEM((CH, D), jnp.float32),
            pltpu.SemaphoreType.DMA,
        ],
    )
    def k(feat_hbm, src_hbm, dst_hbm, out_hbm, agg_sh, sidx, didx, rows, sem):
        cid = lax.axis_index("c")
        sid = lax.axis_index("s")
        wid = sid * NC + cid

        r0 = sid * RPT
        pltpu.sync_copy(feat_hbm.at[pl.ds(r0, RPT)], agg_sh.at[pl.ds(r0, RPT)])
        plsc.subcore_barrier()

        base0 = wid * EPW

        def body(i, carry):
            base = pl.multiple_of(base0 + i * CH, 8)
            pltpu.sync_copy(src_hbm.at[pl.ds(base, CH)], sidx)
            pltpu.sync_copy(dst_hbm.at[pl.ds(base, CH)], didx)
            pltpu.async_copy(feat_hbm.at[sidx], rows, sem).wait()
            pltpu.sync_copy(rows, agg_sh.at[didx], add=True)
            return carry

        lax.fori_loop(0, NCHUNK, body, 0)
        plsc.subcore_barrier()

        o0 = cid * N + r0
        pltpu.sync_copy(agg_sh.at[pl.ds(r0, RPT)], out_hbm.at[pl.ds(o0, RPT)])

    return k(feature, src, dst)


def _tc_linear(p0, p1, feature, W, b2d):
    BLK = 1250

    def body(p0_ref, p1_ref, f_ref, w_ref, b_ref, o_ref):
        h = p0_ref[...] + p1_ref[...] - f_ref[...]
        o_ref[...] = lax.dot_general(
            h, w_ref[...], (((1,), (1,)), ((), ())),
            preferred_element_type=jnp.float32,
        ) + b_ref[...]

    row_spec = pl.BlockSpec((BLK, D), lambda i: (i, 0))
    return pl.pallas_call(
        body,
        grid=(N // BLK,),
        in_specs=[
            row_spec,
            row_spec,
            row_spec,
            pl.BlockSpec((D, D), lambda i: (0, 0)),
            pl.BlockSpec((1, D), lambda i: (0, 0)),
        ],
        out_specs=row_spec,
        out_shape=jax.ShapeDtypeStruct((N, D), jnp.float32),
    )(p0, p1, feature, W, b2d)


@jax.jit
def kernel(feature, edge_index, W, b):
    src = edge_index[0]
    dst = edge_index[1]
    parts = _sc_partials(feature, src, dst)
    p0 = parts[:N]
    p1 = parts[N:]
    return _tc_linear(p0, p1, feature, W, b.reshape(1, D))


# speedup vs baseline: 5.3714x; 5.3714x over previous
import functools

import jax
import jax.numpy as jnp
from jax import lax
from jax.experimental import pallas as pl
from jax.experimental.pallas import tpu as pltpu
from jax.experimental.pallas import tpu_sc as plsc

N = 10000
E = 320000
D = 128

NC = 2
NS = 16
NW = NC * NS
EPW = E // NW
CH = 80
NCHUNK = EPW // CH
RPT = 624
REM = N - NS * RPT
REM_OFF = NS * RPT


def _sc_partials(feature, src, dst):
    mesh = plsc.VectorSubcoreMesh(core_axis_name="c", subcore_axis_name="s")

    @functools.partial(
        pl.kernel,
        mesh=mesh,
        out_type=jax.ShapeDtypeStruct((NC * N, D), jnp.float32),
        scratch_types=[
            pltpu.VMEM_SHARED((N, D), jnp.float32),
            pltpu.VMEM((CH,), jnp.int32),
            pltpu.VMEM((CH,), jnp.int32),
            pltpu.VMEM((CH, D), jnp.float32),
            pltpu.SemaphoreType.DMA,
        ],
    )
    def k(feat_hbm, src_hbm, dst_hbm, out_hbm, agg_sh, sidx, didx, rows, sem):
        cid = lax.axis_index("c")
        sid = lax.axis_index("s")
        wid = sid * NC + cid

        r0 = sid * RPT
        pltpu.sync_copy(feat_hbm.at[pl.ds(r0, RPT)], agg_sh.at[pl.ds(r0, RPT)])

        @pl.when(sid == NS - 1)
        def _():
            pltpu.sync_copy(feat_hbm.at[pl.ds(REM_OFF, REM)],
                            agg_sh.at[pl.ds(REM_OFF, REM)])

        plsc.subcore_barrier()

        base0 = wid * EPW

        def body(i, carry):
            base = pl.multiple_of(base0 + i * CH, 8)
            pltpu.sync_copy(src_hbm.at[pl.ds(base, CH)], sidx)
            pltpu.sync_copy(dst_hbm.at[pl.ds(base, CH)], didx)
            pltpu.async_copy(feat_hbm.at[sidx], rows, sem).wait()
            pltpu.sync_copy(rows, agg_sh.at[didx], add=True)
            return carry

        lax.fori_loop(0, NCHUNK, body, 0)
        plsc.subcore_barrier()

        o0 = cid * N + r0
        pltpu.sync_copy(agg_sh.at[pl.ds(r0, RPT)], out_hbm.at[pl.ds(o0, RPT)])

        @pl.when(sid == NS - 1)
        def _():
            pltpu.sync_copy(agg_sh.at[pl.ds(REM_OFF, REM)],
                            out_hbm.at[pl.ds(cid * N + REM_OFF, REM)])

    return k(feature, src, dst)


def _tc_linear(p0, p1, feature, W, b2d):
    BLK = 1000

    def body(p0_ref, p1_ref, f_ref, w_ref, b_ref, o_ref):
        h = p0_ref[...] + p1_ref[...] - f_ref[...]
        o_ref[...] = lax.dot_general(
            h, w_ref[...], (((1,), (1,)), ((), ())),
            preferred_element_type=jnp.float32,
        ) + b_ref[...]

    row_spec = pl.BlockSpec((BLK, D), lambda i: (i, 0))
    return pl.pallas_call(
        body,
        grid=(N // BLK,),
        in_specs=[
            row_spec,
            row_spec,
            row_spec,
            pl.BlockSpec((D, D), lambda i: (0, 0)),
            pl.BlockSpec((1, D), lambda i: (0, 0)),
        ],
        out_specs=row_spec,
        out_shape=jax.ShapeDtypeStruct((N, D), jnp.float32),
    )(p0, p1, feature, W, b2d)


@jax.jit
def kernel(feature, edge_index, W, b):
    src = edge_index[0]
    dst = edge_index[1]
    parts = _sc_partials(feature, src, dst)
    p0 = parts[:N]
    p1 = parts[N:]
    return _tc_linear(p0, p1, feature, W, b.reshape(1, D))


# speedup vs baseline: 11.2290x; 2.0905x over previous
import functools

import jax
import jax.numpy as jnp
from jax import lax
from jax.experimental import pallas as pl
from jax.experimental.pallas import tpu as pltpu
from jax.experimental.pallas import tpu_sc as plsc

N = 10000
E = 320000
D = 128

NC = 2
NS = 16
NW = NC * NS
EPW = E // NW
CH = 80
NCHUNK = EPW // CH
RPT = 624
REM = N - NS * RPT
REM_OFF = NS * RPT


def _sc_partials(feature, src, dst3):
    mesh = plsc.VectorSubcoreMesh(core_axis_name="c", subcore_axis_name="s")

    @functools.partial(
        pl.kernel,
        mesh=mesh,
        out_type=jax.ShapeDtypeStruct((NC * N, D), jnp.float32),
        scratch_types=[
            pltpu.VMEM_SHARED((N, D), jnp.float32),
            pltpu.VMEM((EPW,), jnp.int32),
            pltpu.VMEM((NCHUNK, CH), jnp.int32),
            pltpu.VMEM((CH, D), jnp.float32),
            pltpu.VMEM((CH, D), jnp.float32),
            pltpu.SemaphoreType.DMA,
            pltpu.SemaphoreType.DMA,
        ],
    )
    def k(feat_hbm, src_hbm, dst_hbm, out_hbm, agg_sh,
          sidx, didx, rows0, rows1, sem0, sem1):
        cid = lax.axis_index("c")
        sid = lax.axis_index("s")
        wid = sid * NC + cid

        pltpu.sync_copy(src_hbm.at[pl.ds(wid * EPW, EPW)], sidx)
        pltpu.sync_copy(dst_hbm.at[wid], didx)
        r0 = sid * RPT
        pltpu.sync_copy(feat_hbm.at[pl.ds(r0, RPT)], agg_sh.at[pl.ds(r0, RPT)])

        @pl.when(sid == NS - 1)
        def _():
            pltpu.sync_copy(feat_hbm.at[pl.ds(REM_OFF, REM)],
                            agg_sh.at[pl.ds(REM_OFF, REM)])

        plsc.subcore_barrier()

        def gather_start(j, rows, sem):
            off = pl.multiple_of(j * CH, 8)
            pltpu.make_async_copy(
                feat_hbm.at[sidx.at[pl.ds(off, CH)]], rows, sem).start()

        def gather_wait(j, rows, sem):
            off = pl.multiple_of(j * CH, 8)
            pltpu.make_async_copy(
                feat_hbm.at[sidx.at[pl.ds(off, CH)]], rows, sem).wait()

        def scatter(j, rows):
            pltpu.sync_copy(rows, agg_sh.at[didx.at[j]], add=True)

        gather_start(0, rows0, sem0)

        def body(j2, carry):
            j = 2 * j2
            gather_start(j + 1, rows1, sem1)
            gather_wait(j, rows0, sem0)
            scatter(j, rows0)
            gather_start(j + 2, rows0, sem0)
            gather_wait(j + 1, rows1, sem1)
            scatter(j + 1, rows1)
            return carry

        lax.fori_loop(0, (NCHUNK - 1) // 2, body, 0)
        gather_wait(NCHUNK - 1, rows0, sem0)
        scatter(NCHUNK - 1, rows0)
        plsc.subcore_barrier()

        o0 = cid * N + r0
        pltpu.sync_copy(agg_sh.at[pl.ds(r0, RPT)], out_hbm.at[pl.ds(o0, RPT)])

        @pl.when(sid == NS - 1)
        def _():
            pltpu.sync_copy(agg_sh.at[pl.ds(REM_OFF, REM)],
                            out_hbm.at[pl.ds(cid * N + REM_OFF, REM)])

    return k(feature, src, dst3)


def _tc_linear(p0, p1, feature, W, b2d):
    BLK = 1000

    def body(p0_ref, p1_ref, f_ref, w_ref, b_ref, o_ref):
        h = p0_ref[...] + p1_ref[...] - f_ref[...]
        o_ref[...] = lax.dot_general(
            h, w_ref[...], (((1,), (1,)), ((), ())),
            preferred_element_type=jnp.float32,
        ) + b_ref[...]

    row_spec = pl.BlockSpec((BLK, D), lambda i: (i, 0))
    return pl.pallas_call(
        body,
        grid=(N // BLK,),
        in_specs=[
            row_spec,
            row_spec,
            row_spec,
            pl.BlockSpec((D, D), lambda i: (0, 0)),
            pl.BlockSpec((1, D), lambda i: (0, 0)),
        ],
        out_specs=row_spec,
        out_shape=jax.ShapeDtypeStruct((N, D), jnp.float32),
    )(p0, p1, feature, W, b2d)


@jax.jit
def kernel(feature, edge_index, W, b):
    src = edge_index[0]
    dst3 = edge_index[1].reshape(NW, NCHUNK, CH)
    parts = _sc_partials(feature, src, dst3)
    p0 = parts[:N]
    p1 = parts[N:]
    return _tc_linear(p0, p1, feature, W, b.reshape(1, D))


# speedup vs baseline: 11.6872x; 1.0408x over previous
import functools

import jax
import jax.numpy as jnp
from jax import lax
from jax.experimental import pallas as pl
from jax.experimental.pallas import tpu as pltpu
from jax.experimental.pallas import tpu_sc as plsc

N = 10000
E = 320000
D = 128

NC = 2
NS = 16
NW = NC * NS
EPW = E // NW
CH = 80
NCHUNK = EPW // CH
RPT = 624
REM = N - NS * RPT
REM_OFF = NS * RPT


def _sc_partials(feature, src, dst3):
    mesh = plsc.VectorSubcoreMesh(core_axis_name="c", subcore_axis_name="s")

    @functools.partial(
        pl.kernel,
        mesh=mesh,
        out_type=jax.ShapeDtypeStruct((NC * N, D), jnp.float32),
        scratch_types=[
            pltpu.VMEM_SHARED((N, D), jnp.float32),
            pltpu.VMEM((EPW,), jnp.int32),
            pltpu.VMEM((NCHUNK, CH), jnp.int32),
            pltpu.VMEM((CH, D), jnp.float32),
            pltpu.VMEM((CH, D), jnp.float32),
            pltpu.SemaphoreType.DMA,
            pltpu.SemaphoreType.DMA,
        ],
    )
    def k(feat_hbm, src_hbm, dst_hbm, out_hbm, agg_sh,
          sidx, didx, rows0, rows1, sem0, sem1):
        cid = lax.axis_index("c")
        sid = lax.axis_index("s")
        wid = sid * NC + cid

        pltpu.sync_copy(src_hbm.at[pl.ds(wid * EPW, EPW)], sidx)
        pltpu.sync_copy(dst_hbm.at[wid], didx)
        r0 = sid * RPT
        pltpu.sync_copy(feat_hbm.at[pl.ds(r0, RPT)], agg_sh.at[pl.ds(r0, RPT)])

        @pl.when(sid == NS - 1)
        def _():
            pltpu.sync_copy(feat_hbm.at[pl.ds(REM_OFF, REM)],
                            agg_sh.at[pl.ds(REM_OFF, REM)])

        plsc.subcore_barrier()

        def gather_start(j, rows, sem):
            off = pl.multiple_of(j * CH, 8)
            pltpu.make_async_copy(
                feat_hbm.at[sidx.at[pl.ds(off, CH)]], rows, sem).start()

        def gather_wait(j, rows, sem):
            off = pl.multiple_of(j * CH, 8)
            pltpu.make_async_copy(
                feat_hbm.at[sidx.at[pl.ds(off, CH)]], rows, sem).wait()

        def scatter(j, rows):
            pltpu.sync_copy(rows, agg_sh.at[didx.at[j]], add=True)

        gather_start(0, rows0, sem0)

        def body(j2, carry):
            j = 2 * j2
            gather_start(j + 1, rows1, sem1)
            gather_wait(j, rows0, sem0)
            scatter(j, rows0)
            gather_start(j + 2, rows0, sem0)
            gather_wait(j + 1, rows1, sem1)
            scatter(j + 1, rows1)
            return carry

        lax.fori_loop(0, (NCHUNK - 1) // 2, body, 0)
        gather_wait(NCHUNK - 1, rows0, sem0)
        scatter(NCHUNK - 1, rows0)
        plsc.subcore_barrier()

        o0 = cid * N + r0
        pltpu.sync_copy(agg_sh.at[pl.ds(r0, RPT)], out_hbm.at[pl.ds(o0, RPT)])

        @pl.when(sid == NS - 1)
        def _():
            pltpu.sync_copy(agg_sh.at[pl.ds(REM_OFF, REM)],
                            out_hbm.at[pl.ds(cid * N + REM_OFF, REM)])

    return k(feature, src, dst3)


def _tc_linear(parts, feature, W, b2d):
    BLK = 1000
    NB = N // BLK

    def body(p0_ref, p1_ref, f_ref, w_ref, b_ref, o_ref):
        h = p0_ref[...] + p1_ref[...] - f_ref[...]
        o_ref[...] = lax.dot_general(
            h, w_ref[...], (((1,), (1,)), ((), ())),
            preferred_element_type=jnp.float32,
        ) + b_ref[...]

    return pl.pallas_call(
        body,
        grid=(NB,),
        in_specs=[
            pl.BlockSpec((BLK, D), lambda i: (i, 0)),
            pl.BlockSpec((BLK, D), lambda i: (i + NB, 0)),
            pl.BlockSpec((BLK, D), lambda i: (i, 0)),
            pl.BlockSpec((D, D), lambda i: (0, 0)),
            pl.BlockSpec((1, D), lambda i: (0, 0)),
        ],
        out_specs=pl.BlockSpec((BLK, D), lambda i: (i, 0)),
        out_shape=jax.ShapeDtypeStruct((N, D), jnp.float32),
    )(parts, parts, feature, W, b2d)


@jax.jit
def kernel(feature, edge_index, W, b):
    src = edge_index[0]
    dst3 = edge_index[1].reshape(NW, NCHUNK, CH)
    parts = _sc_partials(feature, src, dst3)
    return _tc_linear(parts, feature, W, b.reshape(1, D))


# speedup vs baseline: 11.9055x; 1.0187x over previous
import functools

import jax
import jax.numpy as jnp
from jax import lax
from jax.experimental import pallas as pl
from jax.experimental.pallas import tpu as pltpu
from jax.experimental.pallas import tpu_sc as plsc

N = 10000
E = 320000
D = 128

NC = 2
NS = 16
NW = NC * NS
EPW = E // NW
CH = 80
NCHUNK = EPW // CH
RPT = 624
REM = N - NS * RPT
REM_OFF = NS * RPT


def _sc_partials(feature, src, dst3):
    mesh = plsc.VectorSubcoreMesh(core_axis_name="c", subcore_axis_name="s")

    @functools.partial(
        pl.kernel,
        mesh=mesh,
        out_type=jax.ShapeDtypeStruct((NC * N, D), jnp.float32),
        scratch_types=[
            pltpu.VMEM_SHARED((N, D), jnp.float32),
            pltpu.VMEM((EPW,), jnp.int32),
            pltpu.VMEM((NCHUNK, CH), jnp.int32),
            pltpu.VMEM((CH, D), jnp.float32),
            pltpu.VMEM((CH, D), jnp.float32),
            pltpu.SemaphoreType.DMA,
            pltpu.SemaphoreType.DMA,
        ],
    )
    def k(feat_hbm, src_hbm, dst_hbm, out_hbm, agg_sh,
          sidx, didx, rows0, rows1, sem0, sem1):
        cid = lax.axis_index("c")
        sid = lax.axis_index("s")
        wid = sid * NC + cid

        pltpu.sync_copy(src_hbm.at[pl.ds(wid * EPW, EPW)], sidx)
        pltpu.sync_copy(dst_hbm.at[wid], didx)
        r0 = sid * RPT
        pltpu.sync_copy(feat_hbm.at[pl.ds(r0, RPT)], agg_sh.at[pl.ds(r0, RPT)])

        @pl.when(sid == NS - 1)
        def _():
            pltpu.sync_copy(feat_hbm.at[pl.ds(REM_OFF, REM)],
                            agg_sh.at[pl.ds(REM_OFF, REM)])

        plsc.subcore_barrier()

        def gather_start(j, rows, sem):
            off = pl.multiple_of(j * CH, 8)
            pltpu.make_async_copy(
                feat_hbm.at[sidx.at[pl.ds(off, CH)]], rows, sem).start()

        def gather_wait(j, rows, sem):
            off = pl.multiple_of(j * CH, 8)
            pltpu.make_async_copy(
                feat_hbm.at[sidx.at[pl.ds(off, CH)]], rows, sem).wait()

        def scatter(j, rows):
            pltpu.sync_copy(rows, agg_sh.at[didx.at[j]], add=True)

        gather_start(0, rows0, sem0)

        def body(j2, carry):
            j = 2 * j2
            gather_start(j + 1, rows1, sem1)
            gather_wait(j, rows0, sem0)
            scatter(j, rows0)
            gather_start(j + 2, rows0, sem0)
            gather_wait(j + 1, rows1, sem1)
            scatter(j + 1, rows1)
            return carry

        lax.fori_loop(0, (NCHUNK - 1) // 2, body, 0)
        gather_wait(NCHUNK - 1, rows0, sem0)
        scatter(NCHUNK - 1, rows0)
        plsc.subcore_barrier()

        o0 = cid * N + r0
        pltpu.sync_copy(agg_sh.at[pl.ds(r0, RPT)], out_hbm.at[pl.ds(o0, RPT)])

        @pl.when(sid == NS - 1)
        def _():
            pltpu.sync_copy(agg_sh.at[pl.ds(REM_OFF, REM)],
                            out_hbm.at[pl.ds(cid * N + REM_OFF, REM)])

    return k(feature, src, dst3)


def _tc_linear(parts, feature, W, b2d):
    BLK = 2000
    NB = N // BLK

    def body(p0_ref, p1_ref, f_ref, w_ref, b_ref, o_ref):
        h = p0_ref[...] + p1_ref[...] - f_ref[...]
        o_ref[...] = lax.dot_general(
            h, w_ref[...], (((1,), (1,)), ((), ())),
            preferred_element_type=jnp.float32,
        ) + b_ref[...]

    return pl.pallas_call(
        body,
        grid=(NB,),
        in_specs=[
            pl.BlockSpec((BLK, D), lambda i: (i, 0)),
            pl.BlockSpec((BLK, D), lambda i: (i + NB, 0)),
            pl.BlockSpec((BLK, D), lambda i: (i, 0)),
            pl.BlockSpec((D, D), lambda i: (0, 0)),
            pl.BlockSpec((1, D), lambda i: (0, 0)),
        ],
        out_specs=pl.BlockSpec((BLK, D), lambda i: (i, 0)),
        out_shape=jax.ShapeDtypeStruct((N, D), jnp.float32),
    )(parts, parts, feature, W, b2d)


@jax.jit
def kernel(feature, edge_index, W, b):
    src = edge_index[0]
    dst3 = edge_index[1].reshape(NW, NCHUNK, CH)
    parts = _sc_partials(feature, src, dst3)
    return _tc_linear(parts, feature, W, b.reshape(1, D))


# speedup vs baseline: 14.5396x; 1.2213x over previous
import functools

import jax
import jax.numpy as jnp
from jax import lax
from jax.experimental import pallas as pl
from jax.experimental.pallas import tpu as pltpu
from jax.experimental.pallas import tpu_sc as plsc

N = 10000
E = 320000
D = 128

NC = 2
NS = 16
NW = NC * NS
EPW = E // NW
CH = 80
NCHUNK = EPW // CH
RPT = 624
REM = N - NS * RPT
REM_OFF = NS * RPT


def _sc_partials(feature, src, dst):
    mesh = plsc.VectorSubcoreMesh(core_axis_name="c", subcore_axis_name="s")

    @functools.partial(
        pl.kernel,
        mesh=mesh,
        out_type=jax.ShapeDtypeStruct((NC * N, D), jnp.float32),
        scratch_types=[
            pltpu.VMEM_SHARED((N, D), jnp.float32),
            pltpu.VMEM((EPW,), jnp.int32),
            [pltpu.VMEM((CH,), jnp.int32)] * 3,
            [pltpu.VMEM((CH, D), jnp.float32)] * 3,
            [pltpu.SemaphoreType.DMA] * 3,
            [pltpu.SemaphoreType.DMA] * 3,
            pltpu.SemaphoreType.DMA,
        ],
    )
    def k(feat_hbm, src_hbm, dst_hbm, out_hbm, agg_sh,
          sidx, didxs, rowss, gsems, dsems, isem):
        cid = lax.axis_index("c")
        sid = lax.axis_index("s")
        wid = sid * NC + cid
        base0 = wid * EPW
        r0 = sid * RPT

        init_cp = pltpu.make_async_copy(
            feat_hbm.at[pl.ds(r0, RPT)], agg_sh.at[pl.ds(r0, RPT)], isem)
        init_cp.start()
        pltpu.sync_copy(src_hbm.at[pl.ds(base0, EPW)], sidx)

        def start(j, b):
            off = pl.multiple_of(j * CH, 8)
            pltpu.make_async_copy(
                dst_hbm.at[pl.ds(base0 + off, CH)], didxs[b], dsems[b]).start()
            pltpu.make_async_copy(
                feat_hbm.at[sidx.at[pl.ds(off, CH)]], rowss[b], gsems[b]).start()

        def finish(j, b):
            off = pl.multiple_of(j * CH, 8)
            pltpu.make_async_copy(
                feat_hbm.at[sidx.at[pl.ds(off, CH)]], rowss[b], gsems[b]).wait()
            pltpu.make_async_copy(
                dst_hbm.at[pl.ds(base0 + off, CH)], didxs[b], dsems[b]).wait()
            pltpu.sync_copy(rowss[b], agg_sh.at[didxs[b]], add=True)

        start(0, 0)
        start(1, 1)
        init_cp.wait()

        @pl.when(sid == NS - 1)
        def _():
            pltpu.sync_copy(feat_hbm.at[pl.ds(REM_OFF, REM)],
                            agg_sh.at[pl.ds(REM_OFF, REM)])

        plsc.subcore_barrier()

        def body(g, carry):
            j = 3 * g
            start(j + 2, 2)
            finish(j, 0)
            start(j + 3, 0)
            finish(j + 1, 1)
            start(j + 4, 1)
            finish(j + 2, 2)
            return carry

        lax.fori_loop(0, (NCHUNK - 2) // 3, body, 0)
        finish(NCHUNK - 2, 0)
        finish(NCHUNK - 1, 1)
        plsc.subcore_barrier()

        o0 = cid * N + r0
        pltpu.sync_copy(agg_sh.at[pl.ds(r0, RPT)], out_hbm.at[pl.ds(o0, RPT)])

        @pl.when(sid == NS - 1)
        def _():
            pltpu.sync_copy(agg_sh.at[pl.ds(REM_OFF, REM)],
                            out_hbm.at[pl.ds(cid * N + REM_OFF, REM)])

    return k(feature, src, dst)


def _tc_linear(parts, feature, W, b2d):
    BLK = 2000
    NB = N // BLK

    def body(p0_ref, p1_ref, f_ref, w_ref, b_ref, o_ref):
        h = p0_ref[...] + p1_ref[...] - f_ref[...]
        o_ref[...] = lax.dot_general(
            h, w_ref[...], (((1,), (1,)), ((), ())),
            preferred_element_type=jnp.float32,
        ) + b_ref[...]

    return pl.pallas_call(
        body,
        grid=(NB,),
        in_specs=[
            pl.BlockSpec((BLK, D), lambda i: (i, 0)),
            pl.BlockSpec((BLK, D), lambda i: (i + NB, 0)),
            pl.BlockSpec((BLK, D), lambda i: (i, 0)),
            pl.BlockSpec((D, D), lambda i: (0, 0)),
            pl.BlockSpec((1, D), lambda i: (0, 0)),
        ],
        out_specs=pl.BlockSpec((BLK, D), lambda i: (i, 0)),
        out_shape=jax.ShapeDtypeStruct((N, D), jnp.float32),
    )(parts, parts, feature, W, b2d)


@jax.jit
def kernel(feature, edge_index, W, b):
    src = edge_index[0]
    dst = edge_index[1]
    parts = _sc_partials(feature, src, dst)
    return _tc_linear(parts, feature, W, b.reshape(1, D))


# speedup vs baseline: 15.7044x; 1.0801x over previous
import functools

import jax
import jax.numpy as jnp
from jax import lax
from jax.experimental import pallas as pl
from jax.experimental.pallas import tpu as pltpu
from jax.experimental.pallas import tpu_sc as plsc

N = 10000
E = 320000
D = 128

NC = 2
NS = 16
NW = NC * NS
EPW = E // NW
CH = 80
NCHUNK = EPW // CH
RPT = 624
REM = N - NS * RPT
REM_OFF = NS * RPT


def _sc_partials(feature, edges):
    mesh = plsc.VectorSubcoreMesh(core_axis_name="c", subcore_axis_name="s")

    @functools.partial(
        pl.kernel,
        mesh=mesh,
        out_type=jax.ShapeDtypeStruct((NC * N, D), jnp.float32),
        scratch_types=[
            pltpu.VMEM_SHARED((N, D), jnp.float32),
            pltpu.VMEM((EPW,), jnp.int32),
            [pltpu.VMEM((CH,), jnp.int32)] * 3,
            [pltpu.VMEM((CH, D), jnp.float32)] * 3,
            [pltpu.SemaphoreType.DMA] * 3,
            [pltpu.SemaphoreType.DMA] * 3,
            pltpu.SemaphoreType.DMA,
        ],
    )
    def k(feat_hbm, edges_hbm, out_hbm, agg_sh,
          sidx, didxs, rowss, gsems, dsems, isem):
        cid = lax.axis_index("c")
        sid = lax.axis_index("s")
        wid = sid * NC + cid
        base0 = wid * EPW
        r0 = sid * RPT

        init_cp = pltpu.make_async_copy(
            feat_hbm.at[pl.ds(r0, RPT)], agg_sh.at[pl.ds(r0, RPT)], isem)
        init_cp.start()
        pltpu.sync_copy(edges_hbm.at[pl.ds(base0, EPW)], sidx)

        def start(j, b):
            off = pl.multiple_of(j * CH, 8)
            pltpu.make_async_copy(
                edges_hbm.at[pl.ds(E + base0 + off, CH)], didxs[b],
                dsems[b]).start()
            pltpu.make_async_copy(
                feat_hbm.at[sidx.at[pl.ds(off, CH)]], rowss[b], gsems[b]).start()

        def finish(j, b):
            off = pl.multiple_of(j * CH, 8)
            pltpu.make_async_copy(
                feat_hbm.at[sidx.at[pl.ds(off, CH)]], rowss[b], gsems[b]).wait()
            pltpu.make_async_copy(
                edges_hbm.at[pl.ds(E + base0 + off, CH)], didxs[b],
                dsems[b]).wait()
            pltpu.sync_copy(rowss[b], agg_sh.at[didxs[b]], add=True)

        start(0, 0)
        start(1, 1)
        init_cp.wait()

        @pl.when(sid == NS - 1)
        def _():
            pltpu.sync_copy(feat_hbm.at[pl.ds(REM_OFF, REM)],
                            agg_sh.at[pl.ds(REM_OFF, REM)])

        plsc.subcore_barrier()

        def body(g, carry):
            j = 3 * g
            start(j + 2, 2)
            finish(j, 0)
            start(j + 3, 0)
            finish(j + 1, 1)
            start(j + 4, 1)
            finish(j + 2, 2)
            return carry

        lax.fori_loop(0, (NCHUNK - 2) // 3, body, 0)
        finish(NCHUNK - 2, 0)
        finish(NCHUNK - 1, 1)
        plsc.subcore_barrier()

        o0 = cid * N + r0
        pltpu.sync_copy(agg_sh.at[pl.ds(r0, RPT)], out_hbm.at[pl.ds(o0, RPT)])

        @pl.when(sid == NS - 1)
        def _():
            pltpu.sync_copy(agg_sh.at[pl.ds(REM_OFF, REM)],
                            out_hbm.at[pl.ds(cid * N + REM_OFF, REM)])

    return k(feature, edges)


def _tc_linear(parts, feature, W, b2d):
    BLK = 2000
    NB = N // BLK

    def body(p0_ref, p1_ref, f_ref, w_ref, b_ref, o_ref):
        h = p0_ref[...] + p1_ref[...] - f_ref[...]
        o_ref[...] = lax.dot_general(
            h, w_ref[...], (((1,), (1,)), ((), ())),
            preferred_element_type=jnp.float32,
        ) + b_ref[...]

    return pl.pallas_call(
        body,
        grid=(NB,),
        in_specs=[
            pl.BlockSpec((BLK, D), lambda i: (i, 0)),
            pl.BlockSpec((BLK, D), lambda i: (i + NB, 0)),
            pl.BlockSpec((BLK, D), lambda i: (i, 0)),
            pl.BlockSpec((D, D), lambda i: (0, 0)),
            pl.BlockSpec((1, D), lambda i: (0, 0)),
        ],
        out_specs=pl.BlockSpec((BLK, D), lambda i: (i, 0)),
        out_shape=jax.ShapeDtypeStruct((N, D), jnp.float32),
    )(parts, parts, feature, W, b2d)


@jax.jit
def kernel(feature, edge_index, W, b):
    parts = _sc_partials(feature, edge_index.reshape(2 * E))
    return _tc_linear(parts, feature, W, b.reshape(1, D))


# speedup vs baseline: 15.9555x; 1.0160x over previous
import functools

import jax
import jax.numpy as jnp
from jax import lax
from jax.experimental import pallas as pl
from jax.experimental.pallas import tpu as pltpu
from jax.experimental.pallas import tpu_sc as plsc

N = 10000
E = 320000
D = 128

NC = 2
NS = 16
NW = NC * NS
EPW = E // NW
CH = 80
NCHUNK = EPW // CH
RPT = 624
REM = N - NS * RPT
REM_OFF = NS * RPT
ZR = 64


def _sc_partials(feature, edges):
    mesh = plsc.VectorSubcoreMesh(core_axis_name="c", subcore_axis_name="s")

    @functools.partial(
        pl.kernel,
        mesh=mesh,
        out_type=jax.ShapeDtypeStruct((NC * N, D), jnp.float32),
        scratch_types=[
            pltpu.VMEM_SHARED((N, D), jnp.float32),
            pltpu.VMEM((EPW,), jnp.int32),
            [pltpu.VMEM((CH,), jnp.int32)] * 3,
            [pltpu.VMEM((CH, D), jnp.float32)] * 3,
            pltpu.VMEM((ZR, D), jnp.float32),
            [pltpu.SemaphoreType.DMA] * 3,
            [pltpu.SemaphoreType.DMA] * 3,
        ],
    )
    def k(feat_hbm, edges_hbm, out_hbm, agg_sh,
          sidx, didxs, rowss, zbuf, gsems, dsems):
        cid = lax.axis_index("c")
        sid = lax.axis_index("s")
        wid = sid * NC + cid
        base0 = wid * EPW
        r0 = sid * RPT

        pltpu.sync_copy(edges_hbm.at[pl.ds(base0, EPW)], sidx)

        def start(j, b):
            off = pl.multiple_of(j * CH, 8)
            pltpu.make_async_copy(
                edges_hbm.at[pl.ds(E + base0 + off, CH)], didxs[b],
                dsems[b]).start()
            pltpu.make_async_copy(
                feat_hbm.at[sidx.at[pl.ds(off, CH)]], rowss[b], gsems[b]).start()

        def finish(j, b):
            off = pl.multiple_of(j * CH, 8)
            pltpu.make_async_copy(
                feat_hbm.at[sidx.at[pl.ds(off, CH)]], rowss[b], gsems[b]).wait()
            pltpu.make_async_copy(
                edges_hbm.at[pl.ds(E + base0 + off, CH)], didxs[b],
                dsems[b]).wait()
            pltpu.sync_copy(rowss[b], agg_sh.at[didxs[b]], add=True)

        start(0, 0)
        start(1, 1)

        def zbody(r, carry):
            z = jnp.zeros((16,), jnp.float32)
            for c in range(D // 16):
                zbuf[r, pl.ds(c * 16, 16)] = z
            return carry

        lax.fori_loop(0, ZR, zbody, 0)
        for t in range(RPT // ZR):
            pltpu.sync_copy(zbuf, agg_sh.at[pl.ds(r0 + t * ZR, ZR)])
        ZREM = RPT % ZR
        if ZREM:
            pltpu.sync_copy(zbuf.at[pl.ds(0, ZREM)],
                            agg_sh.at[pl.ds(r0 + RPT - ZREM, ZREM)])

        @pl.when(sid == NS - 1)
        def _():
            pltpu.sync_copy(zbuf.at[pl.ds(0, REM)],
                            agg_sh.at[pl.ds(REM_OFF, REM)])

        plsc.subcore_barrier()

        def body(g, carry):
            j = 3 * g
            start(j + 2, 2)
            finish(j, 0)
            start(j + 3, 0)
            finish(j + 1, 1)
            start(j + 4, 1)
            finish(j + 2, 2)
            return carry

        lax.fori_loop(0, (NCHUNK - 2) // 3, body, 0)
        finish(NCHUNK - 2, 0)
        finish(NCHUNK - 1, 1)
        plsc.subcore_barrier()

        o0 = cid * N + r0
        pltpu.sync_copy(agg_sh.at[pl.ds(r0, RPT)], out_hbm.at[pl.ds(o0, RPT)])

        @pl.when(sid == NS - 1)
        def _():
            pltpu.sync_copy(agg_sh.at[pl.ds(REM_OFF, REM)],
                            out_hbm.at[pl.ds(cid * N + REM_OFF, REM)])

    return k(feature, edges)


def _tc_linear(parts, feature, W, b2d):
    BLK = 2000
    NB = N // BLK

    def body(p0_ref, p1_ref, f_ref, w_ref, b_ref, o_ref):
        h = p0_ref[...] + p1_ref[...] + f_ref[...]
        o_ref[...] = lax.dot_general(
            h, w_ref[...], (((1,), (1,)), ((), ())),
            preferred_element_type=jnp.float32,
        ) + b_ref[...]

    return pl.pallas_call(
        body,
        grid=(NB,),
        in_specs=[
            pl.BlockSpec((BLK, D), lambda i: (i, 0)),
            pl.BlockSpec((BLK, D), lambda i: (i + NB, 0)),
            pl.BlockSpec((BLK, D), lambda i: (i, 0)),
            pl.BlockSpec((D, D), lambda i: (0, 0)),
            pl.BlockSpec((1, D), lambda i: (0, 0)),
        ],
        out_specs=pl.BlockSpec((BLK, D), lambda i: (i, 0)),
        out_shape=jax.ShapeDtypeStruct((N, D), jnp.float32),
    )(parts, parts, feature, W, b2d)


@jax.jit
def kernel(feature, edge_index, W, b):
    parts = _sc_partials(feature, edge_index.reshape(2 * E))
    return _tc_linear(parts, feature, W, b.reshape(1, D))
